# Initial kernel scaffold; baseline (speedup 1.0000x reference)
import jax
import jax.numpy as jnp
from jax.experimental import pallas as pl


def kernel(labels, W1, b1, W2, emb_drop, train, unconditioned):
    raise NotImplementedError("write your pallas kernel here")


# speedup vs baseline: 1.2318x; 1.2318x over previous
import functools

import jax
import jax.numpy as jnp
from jax.experimental import pallas as pl
from jax.experimental.pallas import tpu as pltpu

_D = 26
_H = 32
_BT = 1024


def _cond_embed_kernel(u_ref, xT_ref, w1T_ref, b1T_ref, w2T_ref, embdT_ref,
                       outT_ref):
    uncond = u_ref[0] > 0
    xblk = xT_ref[...]
    acc = jnp.zeros(outT_ref.shape, jnp.float32)
    for d in range(_D):
        xrow = xblk[d:d + 1, :]
        nan = jnp.isnan(xrow)
        drop = jnp.logical_or(nan, uncond)
        xsafe = jnp.where(nan, 0.0, xrow)
        w1col = w1T_ref[:, d:d + 1]
        b1col = b1T_ref[:, d:d + 1]
        logits = w1col * xsafe + b1col
        m = jnp.max(logits, axis=0, keepdims=True)
        e = jnp.exp(logits - m)
        s = jnp.sum(e, axis=0, keepdims=True)
        h = e / s
        h = jnp.where(drop, 0.0, h)
        acc = acc + jnp.dot(w2T_ref[d], h, preferred_element_type=jnp.float32)
        acc = acc + jnp.where(drop, 1.0, 0.0) * embdT_ref[:, d:d + 1]
    outT_ref[...] = acc


@functools.partial(jax.jit, static_argnames=())
def kernel(labels, W1, b1, W2, emb_drop, train, unconditioned):
    del train
    B = labels.shape[0]
    xT = labels.T
    w1T = W1[:, 0, :].T
    b1T = b1.T
    w2T = jnp.transpose(W2, (0, 2, 1))
    embdT = emb_drop.T
    u = jnp.asarray(unconditioned, jnp.int32).reshape(1)

    grid = B // _BT
    outT = pl.pallas_call(
        _cond_embed_kernel,
        grid=(grid,),
        in_specs=[
            pl.BlockSpec(memory_space=pltpu.SMEM),
            pl.BlockSpec((_D, _BT), lambda i: (0, i)),
            pl.BlockSpec((_H, _D), lambda i: (0, 0)),
            pl.BlockSpec((_H, _D), lambda i: (0, 0)),
            pl.BlockSpec((_D, _H, _H), lambda i: (0, 0, 0)),
            pl.BlockSpec((_H, _D), lambda i: (0, 0)),
        ],
        out_specs=pl.BlockSpec((_H, _BT), lambda i: (0, i)),
        out_shape=jax.ShapeDtypeStruct((_H, B), jnp.float32),
    )(u, xT, w1T, b1T, w2T, embdT)
    return outT.T
